# baseline (speedup 1.0000x reference)
import jax
import jax.numpy as jnp
from jax import lax
from jax.experimental import pallas as pl
from jax.experimental.pallas import tpu as pltpu

_NUM_VOCAB = 100000
_EMB = 128
_CLS = 100000
_K = 64
_BLK = 10000
_NBLK = _CLS // _BLK


def _gemv_body(a_ref, op_ref, b_ref, aemb_ref, opemb_ref, bemb_ref,
               w_ref, bias_ref, out_ref):
    comb = jnp.concatenate([aemb_ref[0], opemb_ref[0], bemb_ref[0]],
                           axis=1)
    prod = lax.dot_general(comb, w_ref[...],
                           dimension_numbers=(((1,), (1,)), ((), ())),
                           preferred_element_type=jnp.float32)
    out_ref[0] = prod + bias_ref[0]


def _topk_body(proj_ref, sparse_ref, idx_ref, val_ref):
    P = proj_ref[...]
    A = jnp.abs(P)
    abits = lax.bitcast_convert_type(A, jnp.int32)

    def vbit(j, cur):
        cand = cur | (1 << (30 - j))
        cnt = jnp.sum((abits >= cand).astype(jnp.int32))
        return lax.select(cnt >= _K, cand, cur)

    t = lax.fori_loop(0, 31, vbit, jnp.int32(0))

    n_ge = jnp.sum((abits >= t).astype(jnp.int32))
    r_iota = lax.broadcasted_iota(jnp.int32, (_NBLK, _BLK), 0)
    c_iota = lax.broadcasted_iota(jnp.int32, (_NBLK, _BLK), 1)
    gidx = r_iota * _BLK + c_iota

    tie = abits == t

    def fast_cut(_):
        return jnp.int32(_CLS)

    def tie_cut(_):
        n_gt = jnp.sum((abits > t).astype(jnp.int32))
        need = _K - n_gt

        def ibit(j, cur):
            cand = cur | (1 << (16 - j))
            cnt = jnp.sum((tie & (gidx < cand)).astype(jnp.int32))
            return lax.select(cnt < need, cand, cur)

        return lax.fori_loop(0, 17, ibit, jnp.int32(0))

    c_cut = lax.cond(n_ge == _K, fast_cut, tie_cut, None)
    mask = (abits > t) | (tie & (gidx <= c_cut))
    sparse_ref[...] = jnp.where(mask, P, 0.0)

    big = jnp.int32(1 << 28)

    k_iota = lax.broadcasted_iota(jnp.int32, (_NBLK, _K), 1)

    def ext_cond(carry):
        _, _, _, k, total = carry
        return total < _K

    def ext_body(carry):
        cand, st_idx, st_val, k, total = carry
        mrow = jnp.min(cand, axis=1, keepdims=True)
        hit = cand == mrow
        vrow = jnp.sum(jnp.where(hit, P, 0.0), axis=1, keepdims=True)
        got = mrow < big
        st_idx = jnp.where(k_iota == k, jnp.where(got, mrow, big), st_idx)
        st_val = jnp.where(k_iota == k, vrow, st_val)
        cand = jnp.where(hit, big, cand)
        total = total + jnp.sum(got.astype(jnp.int32))
        return cand, st_idx, st_val, k + 1, total

    cand0 = jnp.where(mask, gidx, big)
    _, st_idx, st_val, _, _ = lax.while_loop(
        ext_cond, ext_body,
        (cand0, jnp.full((_NBLK, _K), big, jnp.int32),
         jnp.zeros((_NBLK, _K), jnp.float32), jnp.int32(0), jnp.int32(0)))

    cnt = jnp.sum(mask.astype(jnp.float32), axis=1, keepdims=True)
    ri = lax.broadcasted_iota(jnp.int32, (_NBLK, _NBLK), 0)
    ci = lax.broadcasted_iota(jnp.int32, (_NBLK, _NBLK), 1)
    ltri = (ri > ci).astype(jnp.float32)
    off = lax.dot_general(ltri, cnt, dimension_numbers=(((1,), (0,)), ((), ())),
                          preferred_element_type=jnp.float32)
    slotmat = off.astype(jnp.int32) + k_iota
    valid = st_idx < big

    def gath(j, carry):
        idx_acc, val_acc = carry
        sel = valid & (slotmat == j)
        m = jnp.sum(jnp.where(sel, st_idx, 0))
        v = jnp.sum(jnp.where(sel, st_val, 0.0))
        osl = lax.broadcasted_iota(jnp.int32, (1, _K), 1) == j
        return jnp.where(osl, m, idx_acc), jnp.where(osl, v, val_acc)

    idx_out, val_out = lax.fori_loop(
        0, _K, gath,
        (jnp.zeros((1, _K), jnp.int32), jnp.zeros((1, _K), jnp.float32)))
    idx_ref[...] = idx_out
    val_ref[...] = val_out


def kernel(a, op_idx, b, num_emb, op_emb, W, bias):
    a1 = jnp.reshape(a, (1,)).astype(jnp.int32)
    op1 = jnp.reshape(op_idx, (1,)).astype(jnp.int32)
    b1 = jnp.reshape(b, (1,)).astype(jnp.int32)
    num3 = num_emb.reshape(_NUM_VOCAB, 1, _EMB)
    op3 = op_emb.reshape(-1, 1, _EMB)
    bias3 = bias.reshape(_NBLK, 1, _BLK)

    proj = pl.pallas_call(
        _gemv_body,
        grid_spec=pltpu.PrefetchScalarGridSpec(
            num_scalar_prefetch=3,
            grid=(_NBLK,),
            in_specs=[
                pl.BlockSpec((1, 1, _EMB), lambda i, a_s, o_s, b_s: (a_s[0], 0, 0)),
                pl.BlockSpec((1, 1, _EMB), lambda i, a_s, o_s, b_s: (o_s[0], 0, 0)),
                pl.BlockSpec((1, 1, _EMB), lambda i, a_s, o_s, b_s: (b_s[0], 0, 0)),
                pl.BlockSpec((_BLK, 3 * _EMB), lambda i, a_s, o_s, b_s: (i, 0)),
                pl.BlockSpec((1, 1, _BLK), lambda i, a_s, o_s, b_s: (i, 0, 0)),
            ],
            out_specs=pl.BlockSpec((1, 1, _BLK), lambda i, a_s, o_s, b_s: (i, 0, 0)),
        ),
        out_shape=jax.ShapeDtypeStruct((_NBLK, 1, _BLK), jnp.float32),
    )(a1, op1, b1, num3, op3, num3, W, bias3)

    sparse2, idx2, val2 = pl.pallas_call(
        _topk_body,
        out_shape=(
            jax.ShapeDtypeStruct((_NBLK, _BLK), jnp.float32),
            jax.ShapeDtypeStruct((1, _K), jnp.int32),
            jax.ShapeDtypeStruct((1, _K), jnp.float32),
        ),
    )(proj.reshape(_NBLK, _BLK))
    return sparse2.reshape(_CLS), idx2.reshape(_K), val2.reshape(_K)


# speedup vs baseline: 1.0946x; 1.0946x over previous
import jax
import jax.numpy as jnp
from jax import lax
from jax.experimental import pallas as pl
from jax.experimental.pallas import tpu as pltpu

_NUM_VOCAB = 100000
_EMB = 128
_CLS = 100000
_K = 64
_BLK = 2000
_NBLK = _CLS // _BLK
_NSTREAM = 2
_GRID = _NBLK // _NSTREAM


def _topk(P, sparse_ref, idx_ref, val_ref):
    A = jnp.abs(P)
    abits = lax.bitcast_convert_type(A, jnp.int32)

    def vbit(j, cur):
        cand = cur | (1 << (30 - j))
        cnt = jnp.sum((abits >= cand).astype(jnp.int32))
        return lax.select(cnt >= _K, cand, cur)

    t = lax.fori_loop(0, 31, vbit, jnp.int32(0))

    n_ge = jnp.sum((abits >= t).astype(jnp.int32))
    r_iota = lax.broadcasted_iota(jnp.int32, (_NBLK, _BLK), 0)
    c_iota = lax.broadcasted_iota(jnp.int32, (_NBLK, _BLK), 1)
    gidx = r_iota * _BLK + c_iota
    tie = abits == t

    def fast_cut(_):
        return jnp.int32(_CLS)

    def tie_cut(_):
        n_gt = jnp.sum((abits > t).astype(jnp.int32))
        need = _K - n_gt

        def ibit(j, cur):
            cand = cur | (1 << (16 - j))
            cnt = jnp.sum((tie & (gidx < cand)).astype(jnp.int32))
            return lax.select(cnt < need, cand, cur)

        return lax.fori_loop(0, 17, ibit, jnp.int32(0))

    c_cut = lax.cond(n_ge == _K, fast_cut, tie_cut, None)
    mask = (abits > t) | (tie & (gidx <= c_cut))
    sparse_ref[...] = jnp.where(mask, P, 0.0)

    big = jnp.int32(1 << 28)

    k_iota = lax.broadcasted_iota(jnp.int32, (_NBLK, _K), 1)

    def ext_cond(carry):
        _, _, _, k, total = carry
        return total < _K

    def ext_body(carry):
        cand, st_idx, st_val, k, total = carry
        mrow = jnp.min(cand, axis=1, keepdims=True)
        hit = cand == mrow
        vrow = jnp.sum(jnp.where(hit, P, 0.0), axis=1, keepdims=True)
        got = mrow < big
        st_idx = jnp.where(k_iota == k, jnp.where(got, mrow, big), st_idx)
        st_val = jnp.where(k_iota == k, vrow, st_val)
        cand = jnp.where(hit, big, cand)
        total = total + jnp.sum(got.astype(jnp.int32))
        return cand, st_idx, st_val, k + 1, total

    cand0 = jnp.where(mask, gidx, big)
    _, st_idx, st_val, _, _ = lax.while_loop(
        ext_cond, ext_body,
        (cand0, jnp.full((_NBLK, _K), big, jnp.int32),
         jnp.zeros((_NBLK, _K), jnp.float32), jnp.int32(0), jnp.int32(0)))

    cnt = jnp.sum(mask.astype(jnp.float32), axis=1, keepdims=True)
    ri = lax.broadcasted_iota(jnp.int32, (_NBLK, _NBLK), 0)
    ci = lax.broadcasted_iota(jnp.int32, (_NBLK, _NBLK), 1)
    ltri = (ri > ci).astype(jnp.float32)
    off = lax.dot_general(ltri, cnt, dimension_numbers=(((1,), (0,)), ((), ())),
                          preferred_element_type=jnp.float32)
    slotmat = off.astype(jnp.int32) + k_iota
    valid = st_idx < big

    def gath(j, carry):
        idx_acc, val_acc = carry
        sel = valid & (slotmat == j)
        m = jnp.sum(jnp.where(sel, st_idx, 0))
        v = jnp.sum(jnp.where(sel, st_val, 0.0))
        osl = lax.broadcasted_iota(jnp.int32, (1, _K), 1) == j
        return jnp.where(osl, m, idx_acc), jnp.where(osl, v, val_acc)

    idx_out, val_out = lax.fori_loop(
        0, _K, gath,
        (jnp.zeros((1, _K), jnp.int32), jnp.zeros((1, _K), jnp.float32)))
    idx_ref[...] = idx_out
    val_ref[...] = val_out


def _fused_body(a_ref, op_ref, b_ref, aemb_ref, opemb_ref, bemb_ref,
                *rest):
    w_refs = rest[:_NSTREAM]
    bias_refs = rest[_NSTREAM:2 * _NSTREAM]
    sparse_ref, idx_ref, val_ref, proj_s = rest[2 * _NSTREAM:]
    i = pl.program_id(0)
    comb = jnp.concatenate([aemb_ref[0], opemb_ref[0], bemb_ref[0]],
                           axis=1)
    for s in range(_NSTREAM):
        prod = lax.dot_general(comb, w_refs[s][...],
                               dimension_numbers=(((1,), (1,)), ((), ())),
                               preferred_element_type=jnp.float32)
        proj_s[pl.ds(i + s * _GRID, 1), :] = prod + bias_refs[s][0]

    @pl.when(i == _GRID - 1)
    def _():
        _topk(proj_s[...], sparse_ref, idx_ref, val_ref)


def kernel(a, op_idx, b, num_emb, op_emb, W, bias):
    a1 = jnp.reshape(a, (1,)).astype(jnp.int32)
    op1 = jnp.reshape(op_idx, (1,)).astype(jnp.int32)
    b1 = jnp.reshape(b, (1,)).astype(jnp.int32)
    num3 = num_emb.reshape(_NUM_VOCAB, 1, _EMB)
    op3 = op_emb.reshape(-1, 1, _EMB)
    bias3 = bias.reshape(_NBLK, 1, _BLK)

    w_specs = [
        pl.BlockSpec((_BLK, 3 * _EMB),
                     lambda i, a_s, o_s, b_s, s=s: (i + s * _GRID, 0))
        for s in range(_NSTREAM)
    ]
    bias_specs = [
        pl.BlockSpec((1, 1, _BLK),
                     lambda i, a_s, o_s, b_s, s=s: (i + s * _GRID, 0, 0))
        for s in range(_NSTREAM)
    ]
    sparse2, idx2, val2 = pl.pallas_call(
        _fused_body,
        grid_spec=pltpu.PrefetchScalarGridSpec(
            num_scalar_prefetch=3,
            grid=(_GRID,),
            in_specs=[
                pl.BlockSpec((1, 1, _EMB), lambda i, a_s, o_s, b_s: (a_s[0], 0, 0)),
                pl.BlockSpec((1, 1, _EMB), lambda i, a_s, o_s, b_s: (o_s[0], 0, 0)),
                pl.BlockSpec((1, 1, _EMB), lambda i, a_s, o_s, b_s: (b_s[0], 0, 0)),
                *w_specs,
                *bias_specs,
            ],
            out_specs=[
                pl.BlockSpec((_NBLK, _BLK), lambda i, a_s, o_s, b_s: (0, 0)),
                pl.BlockSpec((1, _K), lambda i, a_s, o_s, b_s: (0, 0)),
                pl.BlockSpec((1, _K), lambda i, a_s, o_s, b_s: (0, 0)),
            ],
            scratch_shapes=[pltpu.VMEM((_NBLK, _BLK), jnp.float32)],
        ),
        out_shape=(
            jax.ShapeDtypeStruct((_NBLK, _BLK), jnp.float32),
            jax.ShapeDtypeStruct((1, _K), jnp.int32),
            jax.ShapeDtypeStruct((1, _K), jnp.float32),
        ),
    )(a1, op1, b1, num3, op3, num3,
      *([W] * _NSTREAM), *([bias3] * _NSTREAM))
    return sparse2.reshape(_CLS), idx2.reshape(_K), val2.reshape(_K)
